# baseline (speedup 1.0000x reference)
import functools

import jax
import jax.numpy as jnp
from jax import lax
from jax.experimental import pallas as pl
from jax.experimental.pallas import tpu as pltpu
from jax.experimental.pallas import tpu_sc as plsc

N_NODES = 10000
N_EDGES = 320000
N_Z = 128
EDGE_DIM = 16

_info = plsc.get_sparse_core_info()
NC, NS = _info.num_cores, _info.num_subcores
NW = NC * NS
EDGES_PER_W = N_EDGES // NW
CHUNK = 1000
N_CHUNKS = EDGES_PER_W // CHUNK


def _tables_body(z_ref, w1_ref, w2_ref, b_ref, t1_ref, t2_ref):
    z = z_ref[...]
    dn = (((1,), (1,)), ((), ()))
    t1_ref[...] = (
        jax.lax.dot_general(z, w1_ref[...], dn, preferred_element_type=jnp.float32)
        + b_ref[...]
    )
    t2_ref[...] = jax.lax.dot_general(
        z, w2_ref[...], dn, preferred_element_type=jnp.float32
    )


def _make_tables(z, W1, W2, b2d):
    return pl.pallas_call(
        _tables_body,
        out_shape=[
            jax.ShapeDtypeStruct((N_NODES, EDGE_DIM), jnp.float32),
            jax.ShapeDtypeStruct((N_NODES, EDGE_DIM), jnp.float32),
        ],
    )(z, W1, W2, b2d)


@functools.partial(
    pl.kernel,
    out_type=jax.ShapeDtypeStruct((N_EDGES * EDGE_DIM,), jnp.float32),
    mesh=plsc.VectorSubcoreMesh(core_axis_name="c", subcore_axis_name="s"),
    compiler_params=pltpu.CompilerParams(use_tc_tiling_on_sc=False),
    scratch_types=[
        pltpu.VMEM((2, CHUNK), jnp.int32),
        pltpu.VMEM((2, CHUNK), jnp.int32),
        pltpu.VMEM((CHUNK, EDGE_DIM), jnp.float32),
        pltpu.VMEM((CHUNK, EDGE_DIM), jnp.float32),
        pltpu.VMEM((CHUNK, EDGE_DIM), jnp.float32),
        pltpu.VMEM((CHUNK, EDGE_DIM), jnp.float32),
        pltpu.VMEM((2, CHUNK * EDGE_DIM), jnp.float32),
        pltpu.SemaphoreType.DMA,
        pltpu.SemaphoreType.DMA,
        pltpu.SemaphoreType.DMA,
        pltpu.SemaphoreType.DMA,
    ],
)
def _edge_gather_add(t1_hbm, t2_hbm, src_hbm, dst_hbm, out_hbm,
                     idx1, idx2, r1a, r2a, r1b, r2b, obuf,
                     sem1a, sem2a, sem1b, sem2b):
    wid = lax.axis_index("s") * NC + lax.axis_index("c")
    base = wid * EDGES_PER_W
    r1 = (r1a, r1b)
    r2 = (r2a, r2b)
    sems = ((sem1a, sem2a), (sem1b, sem2b))

    def issue(c, buf):
        off = base + c * CHUNK
        pltpu.sync_copy(src_hbm.at[pl.ds(off, CHUNK)], idx1.at[buf])
        pltpu.sync_copy(dst_hbm.at[pl.ds(off, CHUNK)], idx2.at[buf])
        cp1 = pltpu.async_copy(t1_hbm.at[idx1.at[buf]], r1[buf], sems[buf][0])
        cp2 = pltpu.async_copy(t2_hbm.at[idx2.at[buf]], r2[buf], sems[buf][1])
        return cp1, cp2

    pending = issue(0, 0)
    for c in range(N_CHUNKS):
        buf = c % 2
        if c + 1 < N_CHUNKS:
            nxt = issue(c + 1, (c + 1) % 2)
        pending[0].wait()
        pending[1].wait()

        def row_body(i, carry, a=r1[buf], b=r2[buf], o=obuf.at[buf]):
            o[pl.ds(i * EDGE_DIM, EDGE_DIM)] = a[i, :] + b[i, :]
            return carry

        lax.fori_loop(0, CHUNK, row_body, 0, unroll=8)
        offf = (base + c * CHUNK) * EDGE_DIM
        pltpu.sync_copy(obuf.at[buf],
                        out_hbm.at[pl.ds(offf, CHUNK * EDGE_DIM)])
        if c + 1 < N_CHUNKS:
            pending = nxt


def kernel(z, edge_index, W, b):
    edge_index = edge_index.astype(jnp.int32)
    W1 = W[:, :N_Z]
    W2 = W[:, N_Z:]
    t1, t2 = _make_tables(z, W1, W2, b.reshape(1, EDGE_DIM))
    flat = _edge_gather_add(t1, t2, edge_index[0], edge_index[1])
    return flat.reshape(N_EDGES, EDGE_DIM)


# speedup vs baseline: 1.0387x; 1.0387x over previous
import functools

import jax
import jax.numpy as jnp
from jax import lax
from jax.experimental import pallas as pl
from jax.experimental.pallas import tpu as pltpu
from jax.experimental.pallas import tpu_sc as plsc

N_NODES = 10000
N_EDGES = 320000
N_Z = 128
EDGE_DIM = 16

_info = plsc.get_sparse_core_info()
NC, NS = _info.num_cores, _info.num_subcores
NW = NC * NS
EDGES_PER_W = N_EDGES // NW
CHUNK = 1000
N_CHUNKS = EDGES_PER_W // CHUNK


def _tables_body(z_ref, w1_ref, w2_ref, b_ref, t1_ref, t2_ref):
    z = z_ref[...]
    dn = (((1,), (1,)), ((), ()))
    t1_ref[...] = (
        jax.lax.dot_general(z, w1_ref[...], dn, preferred_element_type=jnp.float32)
        + b_ref[...]
    )
    t2_ref[...] = jax.lax.dot_general(
        z, w2_ref[...], dn, preferred_element_type=jnp.float32
    )


def _make_tables(z, W1, W2, b2d):
    return pl.pallas_call(
        _tables_body,
        out_shape=[
            jax.ShapeDtypeStruct((N_NODES, EDGE_DIM), jnp.float32),
            jax.ShapeDtypeStruct((N_NODES, EDGE_DIM), jnp.float32),
        ],
    )(z, W1, W2, b2d)


@functools.partial(
    pl.kernel,
    out_type=jax.ShapeDtypeStruct((N_EDGES * EDGE_DIM,), jnp.float32),
    mesh=plsc.VectorSubcoreMesh(core_axis_name="c", subcore_axis_name="s"),
    compiler_params=pltpu.CompilerParams(use_tc_tiling_on_sc=False),
    scratch_types=[
        pltpu.VMEM((2, CHUNK), jnp.int32),
        pltpu.VMEM((2, CHUNK), jnp.int32),
        pltpu.VMEM((CHUNK, EDGE_DIM), jnp.float32),
        pltpu.VMEM((CHUNK, EDGE_DIM), jnp.float32),
        pltpu.VMEM((CHUNK, EDGE_DIM), jnp.float32),
        pltpu.VMEM((CHUNK, EDGE_DIM), jnp.float32),
        pltpu.VMEM((EDGE_DIM * CHUNK,), jnp.float32),
        pltpu.VMEM((EDGE_DIM * CHUNK,), jnp.float32),
        pltpu.SemaphoreType.DMA,
        pltpu.SemaphoreType.DMA,
        pltpu.SemaphoreType.DMA,
        pltpu.SemaphoreType.DMA,
        pltpu.SemaphoreType.DMA,
        pltpu.SemaphoreType.DMA,
    ],
)
def _edge_gather_add(t1_hbm, t2_hbm, ei_hbm, out_hbm,
                     idx1, idx2, r1a, r2a, r1b, r2b, oa, ob,
                     sem1a, sem2a, sem1b, sem2b, semoa, semob):
    wid = lax.axis_index("s") * NC + lax.axis_index("c")
    base = wid * EDGES_PER_W
    r1 = (r1a, r1b)
    r2 = (r2a, r2b)
    sems = ((sem1a, sem2a), (sem1b, sem2b))
    osems = (semoa, semob)
    obufs = (oa, ob)

    def issue(c, buf):
        off = base + c * CHUNK
        pltpu.sync_copy(ei_hbm.at[0, pl.ds(off, CHUNK)], idx1.at[buf])
        pltpu.sync_copy(ei_hbm.at[1, pl.ds(off, CHUNK)], idx2.at[buf])
        cp1 = pltpu.async_copy(t1_hbm.at[idx1.at[buf]], r1[buf], sems[buf][0])
        cp2 = pltpu.async_copy(t2_hbm.at[idx2.at[buf]], r2[buf], sems[buf][1])
        return cp1, cp2

    pending = issue(0, 0)
    for c in range(N_CHUNKS):
        buf = c % 2
        if c + 1 < N_CHUNKS:
            nxt = issue(c + 1, (c + 1) % 2)
        pending[0].wait()
        pending[1].wait()

        def row_body(i, carry, a=r1[buf], b=r2[buf], o=obufs[buf]):
            o[pl.ds(i * EDGE_DIM, EDGE_DIM)] = a[i, :] + b[i, :]
            return carry

        lax.fori_loop(0, CHUNK, row_body, 0, unroll=8)
        offf = (base + c * CHUNK) * EDGE_DIM
        cpo = pltpu.async_copy(
            obufs[buf], out_hbm.at[pl.ds(offf, CHUNK * EDGE_DIM)], osems[buf])
        cpo.wait()
        if c + 1 < N_CHUNKS:
            pending = nxt


def kernel(z, edge_index, W, b):
    edge_index = edge_index.astype(jnp.int32)
    W1 = W[:, :N_Z]
    W2 = W[:, N_Z:]
    t1, t2 = _make_tables(z, W1, W2, b.reshape(1, EDGE_DIM))
    flat = _edge_gather_add(t1, t2, edge_index)
    return flat.reshape(N_EDGES, EDGE_DIM)


# speedup vs baseline: 2.1072x; 2.0286x over previous
import functools

import jax
import jax.numpy as jnp
from jax import lax
from jax.experimental import pallas as pl
from jax.experimental.pallas import tpu as pltpu
from jax.experimental.pallas import tpu_sc as plsc

N_NODES = 10000
N_EDGES = 320000
N_Z = 128
EDGE_DIM = 16

_info = plsc.get_sparse_core_info()
NC, NS = _info.num_cores, _info.num_subcores
NW = NC * NS
EDGES_PER_W = N_EDGES // NW
CHUNK = 400
N_CHUNKS = EDGES_PER_W // CHUNK


def _tables_body(z_ref, w1_ref, w2_ref, b_ref, t1_ref, t2_ref):
    z = z_ref[...]
    dn = (((1,), (1,)), ((), ()))
    t1_ref[...] = (
        jax.lax.dot_general(z, w1_ref[...], dn, preferred_element_type=jnp.float32)
        + b_ref[...]
    )
    t2_ref[...] = jax.lax.dot_general(
        z, w2_ref[...], dn, preferred_element_type=jnp.float32
    )


def _make_tables(z, W1, W2, b2d):
    return pl.pallas_call(
        _tables_body,
        out_shape=[
            jax.ShapeDtypeStruct((N_NODES, EDGE_DIM), jnp.float32),
            jax.ShapeDtypeStruct((N_NODES, EDGE_DIM), jnp.float32),
        ],
    )(z, W1, W2, b2d)


@functools.partial(
    pl.kernel,
    out_type=jax.ShapeDtypeStruct((EDGE_DIM, N_EDGES), jnp.float32),
    mesh=plsc.VectorSubcoreMesh(core_axis_name="c", subcore_axis_name="s"),
    compiler_params=pltpu.CompilerParams(use_tc_tiling_on_sc=False),
    scratch_types=[
        pltpu.VMEM((2, CHUNK), jnp.int32),
        pltpu.VMEM((2, CHUNK), jnp.int32),
        pltpu.VMEM((CHUNK, EDGE_DIM), jnp.float32),
        pltpu.VMEM((CHUNK, EDGE_DIM), jnp.float32),
        pltpu.VMEM((CHUNK, EDGE_DIM), jnp.float32),
        pltpu.VMEM((CHUNK, EDGE_DIM), jnp.float32),
        pltpu.VMEM((EDGE_DIM, CHUNK), jnp.float32),
        pltpu.VMEM((EDGE_DIM, CHUNK), jnp.float32),
        pltpu.SemaphoreType.DMA,
        pltpu.SemaphoreType.DMA,
        pltpu.SemaphoreType.DMA,
        pltpu.SemaphoreType.DMA,
        pltpu.SemaphoreType.DMA,
        pltpu.SemaphoreType.DMA,
    ],
)
def _edge_gather_add(t1_hbm, t2_hbm, ei_hbm, out_hbm,
                     idx1, idx2, r1a, r2a, r1b, r2b, oa, ob,
                     sem1a, sem2a, sem1b, sem2b, semoa, semob):
    wid = lax.axis_index("s") * NC + lax.axis_index("c")
    base = wid * EDGES_PER_W
    r1 = (r1a, r1b)
    r2 = (r2a, r2b)
    obufs = (oa, ob)
    sems = ((sem1a, sem2a), (sem1b, sem2b))
    osems = (semoa, semob)

    lane = lax.iota(jnp.int32, EDGE_DIM)
    perm_m = {d: (lane - d) & 15 for d in (1, 2, 4, 8)}
    perm_p = {d: (lane + d) & 15 for d in (1, 2, 4, 8)}
    masks = {d: (lane & d) == 0 for d in (1, 2, 4, 8)}

    def issue(c, buf):
        off = base + c * CHUNK
        pltpu.sync_copy(ei_hbm.at[0, pl.ds(off, CHUNK)], idx1.at[buf])
        pltpu.sync_copy(ei_hbm.at[1, pl.ds(off, CHUNK)], idx2.at[buf])
        cp1 = pltpu.async_copy(t1_hbm.at[idx1.at[buf]], r1[buf], sems[buf][0])
        cp2 = pltpu.async_copy(t2_hbm.at[idx2.at[buf]], r2[buf], sems[buf][1])
        return cp1, cp2

    pending = issue(0, 0)
    for c in range(N_CHUNKS):
        buf = c % 2
        if c + 1 < N_CHUNKS:
            nxt = issue(c + 1, (c + 1) % 2)
        pending[0].wait()
        pending[1].wait()

        def blk_body(blk, carry, a=r1[buf], b=r2[buf], o=obufs[buf]):
            j0 = blk * EDGE_DIM
            rows = [a[j0 + k, :] + b[j0 + k, :] for k in range(EDGE_DIM)]
            for d in (8, 4, 2, 1):
                for i in range(EDGE_DIM):
                    if i & d:
                        continue
                    x, y = rows[i], rows[i | d]
                    rows[i] = jnp.where(masks[d], x, y[perm_m[d]])
                    rows[i | d] = jnp.where(masks[d], x[perm_p[d]], y)
            for k in range(EDGE_DIM):
                o[k, pl.ds(j0, EDGE_DIM)] = rows[k]
            return carry

        lax.fori_loop(0, CHUNK // EDGE_DIM, blk_body, 0)
        cpo = pltpu.async_copy(
            obufs[buf],
            out_hbm.at[:, pl.ds(base + c * CHUNK, CHUNK)],
            osems[buf],
        )
        cpo.wait()
        if c + 1 < N_CHUNKS:
            pending = nxt


def kernel(z, edge_index, W, b):
    edge_index = edge_index.astype(jnp.int32)
    W1 = W[:, :N_Z]
    W2 = W[:, N_Z:]
    t1, t2 = _make_tables(z, W1, W2, b.reshape(1, EDGE_DIM))
    out_t = _edge_gather_add(t1, t2, edge_index)
    return out_t.T


# speedup vs baseline: 2.6112x; 1.2392x over previous
import functools

import jax
import jax.numpy as jnp
from jax import lax
from jax.experimental import pallas as pl
from jax.experimental.pallas import tpu as pltpu
from jax.experimental.pallas import tpu_sc as plsc

N_NODES = 10000
N_EDGES = 320000
N_Z = 128
EDGE_DIM = 16

_info = plsc.get_sparse_core_info()
NC, NS = _info.num_cores, _info.num_subcores
NW = NC * NS
EDGES_PER_W = N_EDGES // NW
CHUNK = 400
N_CHUNKS = EDGES_PER_W // CHUNK


def _tables_body(z_ref, w1_ref, w2_ref, b_ref, t1_ref, t2_ref):
    z = z_ref[...]
    dn = (((1,), (1,)), ((), ()))
    t1_ref[...] = (
        jax.lax.dot_general(z, w1_ref[...], dn, preferred_element_type=jnp.float32)
        + b_ref[...]
    )
    t2_ref[...] = jax.lax.dot_general(
        z, w2_ref[...], dn, preferred_element_type=jnp.float32
    )


def _make_tables(z, W1, W2, b2d):
    return pl.pallas_call(
        _tables_body,
        out_shape=[
            jax.ShapeDtypeStruct((N_NODES, EDGE_DIM), jnp.float32),
            jax.ShapeDtypeStruct((N_NODES, EDGE_DIM), jnp.float32),
        ],
    )(z, W1, W2, b2d)


@functools.partial(
    pl.kernel,
    out_type=jax.ShapeDtypeStruct((EDGE_DIM, N_EDGES), jnp.float32),
    mesh=plsc.VectorSubcoreMesh(core_axis_name="c", subcore_axis_name="s"),
    compiler_params=pltpu.CompilerParams(use_tc_tiling_on_sc=False),
    scratch_types=[
        pltpu.VMEM((2, EDGES_PER_W), jnp.int32),
        pltpu.VMEM((CHUNK, EDGE_DIM), jnp.float32),
        pltpu.VMEM((CHUNK, EDGE_DIM), jnp.float32),
        pltpu.VMEM((CHUNK, EDGE_DIM), jnp.float32),
        pltpu.VMEM((CHUNK, EDGE_DIM), jnp.float32),
        pltpu.VMEM((EDGE_DIM, CHUNK), jnp.float32),
        pltpu.VMEM((EDGE_DIM, CHUNK), jnp.float32),
        pltpu.SemaphoreType.DMA,
        pltpu.SemaphoreType.DMA,
        pltpu.SemaphoreType.DMA,
        pltpu.SemaphoreType.DMA,
        pltpu.SemaphoreType.DMA,
        pltpu.SemaphoreType.DMA,
    ],
)
def _edge_gather_add(t1_hbm, t2_hbm, ei_hbm, out_hbm,
                     idxs, r1a, r2a, r1b, r2b, oa, ob,
                     sem1a, sem2a, sem1b, sem2b, semoa, semob):
    wid = lax.axis_index("s") * NC + lax.axis_index("c")
    base = wid * EDGES_PER_W
    r1 = (r1a, r1b)
    r2 = (r2a, r2b)
    obufs = (oa, ob)
    sems = ((sem1a, sem2a), (sem1b, sem2b))
    osems = (semoa, semob)

    lane = lax.iota(jnp.int32, EDGE_DIM)
    perm_m = {d: (lane - d) & 15 for d in (1, 2, 4, 8)}
    perm_p = {d: (lane + d) & 15 for d in (1, 2, 4, 8)}
    masks = {d: (lane & d) == 0 for d in (1, 2, 4, 8)}

    pltpu.sync_copy(ei_hbm.at[0, pl.ds(base, EDGES_PER_W)], idxs.at[0])
    pltpu.sync_copy(ei_hbm.at[1, pl.ds(base, EDGES_PER_W)], idxs.at[1])

    def issue(c, buf):
        sl = pl.ds(c * CHUNK, CHUNK)
        cp1 = pltpu.async_copy(t1_hbm.at[idxs.at[0, sl]], r1[buf], sems[buf][0])
        cp2 = pltpu.async_copy(t2_hbm.at[idxs.at[1, sl]], r2[buf], sems[buf][1])
        return cp1, cp2

    pending = issue(0, 0)
    pend_o = [None, None]
    for c in range(N_CHUNKS):
        buf = c % 2
        if c + 1 < N_CHUNKS:
            nxt = issue(c + 1, (c + 1) % 2)
        pending[0].wait()
        pending[1].wait()
        if pend_o[buf] is not None:
            pend_o[buf].wait()

        def blk_body(blk, carry, a=r1[buf], b=r2[buf], o=obufs[buf]):
            j0 = blk * EDGE_DIM
            rows = [a[j0 + k, :] + b[j0 + k, :] for k in range(EDGE_DIM)]
            for d in (8, 4, 2, 1):
                for i in range(EDGE_DIM):
                    if i & d:
                        continue
                    x, y = rows[i], rows[i | d]
                    rows[i] = jnp.where(masks[d], x, y[perm_m[d]])
                    rows[i | d] = jnp.where(masks[d], x[perm_p[d]], y)
            for k in range(EDGE_DIM):
                o[k, pl.ds(j0, EDGE_DIM)] = rows[k]
            return carry

        lax.fori_loop(0, CHUNK // EDGE_DIM, blk_body, 0)
        pend_o[buf] = pltpu.async_copy(
            obufs[buf],
            out_hbm.at[:, pl.ds(base + c * CHUNK, CHUNK)],
            osems[buf],
        )
        if c + 1 < N_CHUNKS:
            pending = nxt
    pend_o[0].wait()
    if pend_o[1] is not None:
        pend_o[1].wait()


def kernel(z, edge_index, W, b):
    edge_index = edge_index.astype(jnp.int32)
    W1 = W[:, :N_Z]
    W2 = W[:, N_Z:]
    t1, t2 = _make_tables(z, W1, W2, b.reshape(1, EDGE_DIM))
    out_t = _edge_gather_add(t1, t2, edge_index)
    return out_t.T


# speedup vs baseline: 2.7113x; 1.0383x over previous
import functools

import jax
import jax.numpy as jnp
from jax import lax
from jax.experimental import pallas as pl
from jax.experimental.pallas import tpu as pltpu
from jax.experimental.pallas import tpu_sc as plsc

N_NODES = 10000
N_EDGES = 320000
N_Z = 128
EDGE_DIM = 16

_info = plsc.get_sparse_core_info()
NC, NS = _info.num_cores, _info.num_subcores
NW = NC * NS
EDGES_PER_W = N_EDGES // NW
CHUNK = 400
N_CHUNKS = EDGES_PER_W // CHUNK


def _tables_body(z_ref, w1_ref, w2_ref, b_ref, ei_ref,
                 t1_ref, t2_ref, src_ref, dst_ref):
    z = z_ref[...]
    dn = (((1,), (1,)), ((), ()))
    t1_ref[...] = (
        jax.lax.dot_general(z, w1_ref[...], dn, preferred_element_type=jnp.float32)
        + b_ref[...]
    )
    t2_ref[...] = jax.lax.dot_general(
        z, w2_ref[...], dn, preferred_element_type=jnp.float32
    )
    src_ref[...] = ei_ref[0, :]
    dst_ref[...] = ei_ref[1, :]


def _make_tables(z, W1, W2, b2d, edge_index):
    return pl.pallas_call(
        _tables_body,
        out_shape=[
            jax.ShapeDtypeStruct((N_NODES, EDGE_DIM), jnp.float32),
            jax.ShapeDtypeStruct((N_NODES, EDGE_DIM), jnp.float32),
            jax.ShapeDtypeStruct((N_EDGES,), jnp.int32),
            jax.ShapeDtypeStruct((N_EDGES,), jnp.int32),
        ],
    )(z, W1, W2, b2d, edge_index)


@functools.partial(
    pl.kernel,
    out_type=jax.ShapeDtypeStruct((EDGE_DIM, N_EDGES), jnp.float32),
    mesh=plsc.VectorSubcoreMesh(core_axis_name="c", subcore_axis_name="s"),
    compiler_params=pltpu.CompilerParams(use_tc_tiling_on_sc=False),
    scratch_types=[
        pltpu.VMEM((2, EDGES_PER_W), jnp.int32),
        pltpu.VMEM((CHUNK, EDGE_DIM), jnp.float32),
        pltpu.VMEM((CHUNK, EDGE_DIM), jnp.float32),
        pltpu.VMEM((CHUNK, EDGE_DIM), jnp.float32),
        pltpu.VMEM((CHUNK, EDGE_DIM), jnp.float32),
        pltpu.VMEM((EDGE_DIM, CHUNK), jnp.float32),
        pltpu.VMEM((EDGE_DIM, CHUNK), jnp.float32),
        pltpu.SemaphoreType.DMA,
        pltpu.SemaphoreType.DMA,
        pltpu.SemaphoreType.DMA,
        pltpu.SemaphoreType.DMA,
        pltpu.SemaphoreType.DMA,
        pltpu.SemaphoreType.DMA,
    ],
)
def _edge_gather_add(t1_hbm, t2_hbm, src_hbm, dst_hbm, out_hbm,
                     idxs, r1a, r2a, r1b, r2b, oa, ob,
                     sem1a, sem2a, sem1b, sem2b, semoa, semob):
    wid = lax.axis_index("s") * NC + lax.axis_index("c")
    base = wid * EDGES_PER_W
    r1 = (r1a, r1b)
    r2 = (r2a, r2b)
    obufs = (oa, ob)
    sems = ((sem1a, sem2a), (sem1b, sem2b))
    osems = (semoa, semob)

    lane = lax.iota(jnp.int32, EDGE_DIM)
    perm_m = {d: (lane - d) & 15 for d in (1, 2, 4, 8)}
    perm_p = {d: (lane + d) & 15 for d in (1, 2, 4, 8)}
    masks = {d: (lane & d) == 0 for d in (1, 2, 4, 8)}

    pltpu.sync_copy(src_hbm.at[pl.ds(base, EDGES_PER_W)], idxs.at[0])
    pltpu.sync_copy(dst_hbm.at[pl.ds(base, EDGES_PER_W)], idxs.at[1])

    def issue(c, buf):
        sl = pl.ds(c * CHUNK, CHUNK)
        cp1 = pltpu.async_copy(t1_hbm.at[idxs.at[0, sl]], r1[buf], sems[buf][0])
        cp2 = pltpu.async_copy(t2_hbm.at[idxs.at[1, sl]], r2[buf], sems[buf][1])
        return cp1, cp2

    pending = issue(0, 0)
    pend_o = [None, None]
    for c in range(N_CHUNKS):
        buf = c % 2
        if c + 1 < N_CHUNKS:
            nxt = issue(c + 1, (c + 1) % 2)
        pending[0].wait()
        pending[1].wait()
        if pend_o[buf] is not None:
            pend_o[buf].wait()

        def blk_body(blk, carry, a=r1[buf], b=r2[buf], o=obufs[buf]):
            j0 = blk * EDGE_DIM
            rows = [a[j0 + k, :] + b[j0 + k, :] for k in range(EDGE_DIM)]
            for d in (8, 4, 2, 1):
                for i in range(EDGE_DIM):
                    if i & d:
                        continue
                    x, y = rows[i], rows[i | d]
                    rows[i] = jnp.where(masks[d], x, y[perm_m[d]])
                    rows[i | d] = jnp.where(masks[d], x[perm_p[d]], y)
            for k in range(EDGE_DIM):
                o[k, pl.ds(j0, EDGE_DIM)] = rows[k]
            return carry

        lax.fori_loop(0, CHUNK // EDGE_DIM, blk_body, 0)
        pend_o[buf] = pltpu.async_copy(
            obufs[buf],
            out_hbm.at[:, pl.ds(base + c * CHUNK, CHUNK)],
            osems[buf],
        )
        if c + 1 < N_CHUNKS:
            pending = nxt
    pend_o[0].wait()
    if pend_o[1] is not None:
        pend_o[1].wait()


def kernel(z, edge_index, W, b):
    edge_index = edge_index.astype(jnp.int32)
    W1 = W[:, :N_Z]
    W2 = W[:, N_Z:]
    t1, t2, src, dst = _make_tables(z, W1, W2, b.reshape(1, EDGE_DIM),
                                    edge_index)
    out_t = _edge_gather_add(t1, t2, src, dst)
    return out_t.T
